# Initial kernel scaffold; baseline (speedup 1.0000x reference)
import jax
import jax.numpy as jnp
from jax.experimental import pallas as pl


def kernel(x, adj, lt):
    raise NotImplementedError("write your pallas kernel here")


# speedup vs baseline: 1.2333x; 1.2333x over previous
import jax
import jax.numpy as jnp
from jax.experimental import pallas as pl

N_ROWS = 1000000
BLOCK_ROWS = 2000


def _concat_body(lt_ref, x_ref, out_ref):
    out_ref[:, 0:64] = lt_ref[...]
    out_ref[:, 64:128] = x_ref[...]


def kernel(x, adj, lt):
    del adj
    n = lt.shape[0]
    grid = (n // BLOCK_ROWS,)
    return pl.pallas_call(
        _concat_body,
        grid=grid,
        in_specs=[
            pl.BlockSpec((BLOCK_ROWS, 64), lambda i: (i, 0)),
            pl.BlockSpec((BLOCK_ROWS, 64), lambda i: (i, 0)),
        ],
        out_specs=pl.BlockSpec((BLOCK_ROWS, 128), lambda i: (i, 0)),
        out_shape=jax.ShapeDtypeStruct((n, 128), jnp.float32),
    )(lt, x)


# speedup vs baseline: 1.3987x; 1.1341x over previous
import jax
import jax.numpy as jnp
from jax.experimental import pallas as pl

N_ROWS = 1000000
BLOCK_ROWS = 8000


def _concat_body(lt_ref, x_ref, out_ref):
    out_ref[:, 0:64] = lt_ref[...]
    out_ref[:, 64:128] = x_ref[...]


def kernel(x, adj, lt):
    del adj
    n = lt.shape[0]
    grid = (n // BLOCK_ROWS,)
    return pl.pallas_call(
        _concat_body,
        grid=grid,
        in_specs=[
            pl.BlockSpec((BLOCK_ROWS, 64), lambda i: (i, 0)),
            pl.BlockSpec((BLOCK_ROWS, 64), lambda i: (i, 0)),
        ],
        out_specs=pl.BlockSpec((BLOCK_ROWS, 128), lambda i: (i, 0)),
        out_shape=jax.ShapeDtypeStruct((n, 128), jnp.float32),
    )(lt, x)


# speedup vs baseline: 1.4007x; 1.0014x over previous
import jax
import jax.numpy as jnp
from jax.experimental import pallas as pl

N_ROWS = 1000000
BLOCK_ROWS = 20000


def _concat_body(lt_ref, x_ref, out_ref):
    out_ref[:, 0:64] = lt_ref[...]
    out_ref[:, 64:128] = x_ref[...]


def kernel(x, adj, lt):
    del adj
    n = lt.shape[0]
    grid = (n // BLOCK_ROWS,)
    return pl.pallas_call(
        _concat_body,
        grid=grid,
        in_specs=[
            pl.BlockSpec((BLOCK_ROWS, 64), lambda i: (i, 0)),
            pl.BlockSpec((BLOCK_ROWS, 64), lambda i: (i, 0)),
        ],
        out_specs=pl.BlockSpec((BLOCK_ROWS, 128), lambda i: (i, 0)),
        out_shape=jax.ShapeDtypeStruct((n, 128), jnp.float32),
    )(lt, x)
